# baseline (speedup 1.0000x reference)
import jax
import jax.numpy as jnp
from jax.experimental import pallas as pl
from jax.experimental.pallas import tpu as pltpu

_N = 4096
_F = 256
_BM = 512
_BO = 1024
_NBLK = _N // _BM
_NOUT = _N // _BO
_H = _N // 2


def _cheb_kernel(L_ref, xb_ref, w_ref, b_ref, out_ref, Lb_ref, y_ref, acc_ref):
    m = pl.program_id(0)

    @pl.when(m < _NBLK)
    def _stream():
        row = pl.ds(m * _BM, _BM)
        strip = L_ref[...].astype(jnp.bfloat16)
        Lb_ref[row, :] = strip
        t1m = jnp.dot(strip, xb_ref[...], preferred_element_type=jnp.float32)
        t1mb = t1m.astype(jnp.bfloat16)
        w0m2 = w_ref[0, :, :] - w_ref[2, :, :]
        w1 = w_ref[1, :, :]
        w2 = w_ref[2, :, :]
        y = jnp.dot(t1mb, w2, preferred_element_type=jnp.float32)
        y_ref[row, :] = (2.0 * y).astype(jnp.bfloat16)
        accm = jnp.dot(xb_ref[row, :], w0m2,
                       preferred_element_type=jnp.float32)
        accm += jnp.dot(t1mb, w1, preferred_element_type=jnp.float32)
        acc_ref[row, :] = (accm + b_ref[...]).astype(jnp.bfloat16)

    @pl.when((m >= _NBLK - _NOUT) & (m < _NBLK))
    def _left_half():
        rows = pl.ds((m - (_NBLK - _NOUT)) * _BO, _BO)
        left = jnp.dot(Lb_ref[rows, :_H], y_ref[:_H, :],
                       preferred_element_type=jnp.float32)
        acc_ref[rows, :] = (acc_ref[rows, :].astype(jnp.float32)
                            + left).astype(jnp.bfloat16)

    @pl.when(m >= _NBLK)
    def _produce():
        rows = pl.ds((m - _NBLK) * _BO, _BO)
        right = jnp.dot(Lb_ref[rows, _H:], y_ref[_H:, :],
                        preferred_element_type=jnp.float32)
        out_ref[...] = right + acc_ref[rows, :].astype(jnp.float32)


def kernel(x, L_tilde, weight, bias):
    xb = x.astype(jnp.bfloat16)
    wb = weight.astype(jnp.bfloat16)
    bias2d = bias.reshape(1, _F)

    out = pl.pallas_call(
        _cheb_kernel,
        grid=(_NBLK + _NOUT,),
        in_specs=[
            pl.BlockSpec(
                (_BM, _N),
                lambda m: (jnp.minimum(m, _NBLK - 1), 0)),
            pl.BlockSpec((_N, _F), lambda m: (0, 0)),
            pl.BlockSpec((3, _F, _F), lambda m: (0, 0, 0)),
            pl.BlockSpec((1, _F), lambda m: (0, 0)),
        ],
        out_specs=pl.BlockSpec(
            (_BO, _F),
            lambda m: (jnp.maximum(m - _NBLK, 0), 0)),
        out_shape=jax.ShapeDtypeStruct((_N, _F), jnp.float32),
        scratch_shapes=[
            pltpu.VMEM((_N, _N), jnp.bfloat16),
            pltpu.VMEM((_N, _F), jnp.bfloat16),
            pltpu.VMEM((_N, _F), jnp.bfloat16),
        ],
        compiler_params=pltpu.CompilerParams(
            dimension_semantics=("arbitrary",),
        ),
    )(L_tilde, xb, wb, bias2d)
    return out


# speedup vs baseline: 1.0153x; 1.0153x over previous
import jax
import jax.numpy as jnp
from jax.experimental import pallas as pl
from jax.experimental.pallas import tpu as pltpu

_N = 4096
_F = 256
_BM = 512
_BO = 1024
_NBLK = _N // _BM
_NOUT = _N // _BO


def _cheb_kernel(L_ref, xb_ref, w_ref, b_ref, out_ref, Lb_ref, t1_ref, y_ref):
    m = pl.program_id(0)

    @pl.when(m < _NBLK)
    def _stream():
        row = pl.ds(m * _BM, _BM)
        strip = L_ref[...].astype(jnp.bfloat16)
        Lb_ref[row, :] = strip
        t1 = jnp.dot(strip, xb_ref[...], preferred_element_type=jnp.float32)
        t1_ref[row, :] = t1.astype(jnp.bfloat16)

    @pl.when(m == _NBLK)
    def _fold_w2():
        w2 = w_ref[2, :, :].astype(jnp.bfloat16)
        y = jnp.dot(t1_ref[...], w2, preferred_element_type=jnp.float32)
        y_ref[...] = (2.0 * y).astype(jnp.bfloat16)

    @pl.when(m >= _NBLK)
    def _produce():
        row = pl.ds((m - _NBLK) * _BO, _BO)
        w0m2 = (w_ref[0, :, :] - w_ref[2, :, :]).astype(jnp.bfloat16)
        w1 = w_ref[1, :, :].astype(jnp.bfloat16)
        acc = jnp.dot(Lb_ref[row, :], y_ref[...],
                      preferred_element_type=jnp.float32)
        acc += jnp.dot(xb_ref[row, :], w0m2, preferred_element_type=jnp.float32)
        acc += jnp.dot(t1_ref[row, :], w1, preferred_element_type=jnp.float32)
        out_ref[...] = acc + b_ref[...]


def kernel(x, L_tilde, weight, bias):
    xb = x.astype(jnp.bfloat16)
    bias2d = bias.reshape(1, _F)

    out = pl.pallas_call(
        _cheb_kernel,
        grid=(_NBLK + _NOUT,),
        in_specs=[
            pl.BlockSpec(
                (_BM, _N),
                lambda m: (jnp.minimum(m, _NBLK - 1), 0)),
            pl.BlockSpec((_N, _F), lambda m: (0, 0)),
            pl.BlockSpec((3, _F, _F), lambda m: (0, 0, 0)),
            pl.BlockSpec((1, _F), lambda m: (0, 0)),
        ],
        out_specs=pl.BlockSpec(
            (_BO, _F),
            lambda m: (jnp.maximum(m - _NBLK, 0), 0)),
        out_shape=jax.ShapeDtypeStruct((_N, _F), jnp.float32),
        scratch_shapes=[
            pltpu.VMEM((_N, _N), jnp.bfloat16),
            pltpu.VMEM((_N, _F), jnp.bfloat16),
            pltpu.VMEM((_N, _F), jnp.bfloat16),
        ],
        compiler_params=pltpu.CompilerParams(
            dimension_semantics=("arbitrary",),
        ),
    )(L_tilde, xb, weight, bias2d)
    return out


# speedup vs baseline: 1.0361x; 1.0204x over previous
import jax
import jax.numpy as jnp
from jax.experimental import pallas as pl
from jax.experimental.pallas import tpu as pltpu

_N = 4096
_F = 256
_BM = 512
_BO = 1024
_NBLK = _N // _BM
_NOUT = _N // _BO


def _cheb_kernel(L_ref, xb_ref, w_ref, b_ref, out_ref, Lb_ref, y_ref, acc_ref):
    m = pl.program_id(0)

    @pl.when(m < _NBLK)
    def _stream():
        row = pl.ds(m * _BM, _BM)
        strip = L_ref[...].astype(jnp.bfloat16)
        Lb_ref[row, :] = strip
        t1m = jnp.dot(strip, xb_ref[...], preferred_element_type=jnp.float32)
        t1mb = t1m.astype(jnp.bfloat16)
        w0m2 = (w_ref[0, :, :] - w_ref[2, :, :]).astype(jnp.bfloat16)
        w1 = w_ref[1, :, :].astype(jnp.bfloat16)
        w2 = w_ref[2, :, :].astype(jnp.bfloat16)
        y = jnp.dot(t1mb, w2, preferred_element_type=jnp.float32)
        y_ref[row, :] = (2.0 * y).astype(jnp.bfloat16)
        accm = jnp.dot(xb_ref[row, :], w0m2,
                       preferred_element_type=jnp.float32)
        accm += jnp.dot(t1mb, w1, preferred_element_type=jnp.float32)
        acc_ref[row, :] = (accm + b_ref[...]).astype(jnp.bfloat16)

    @pl.when(m >= _NBLK)
    def _produce():
        rows = pl.ds((m - _NBLK) * _BO, _BO)
        a = jnp.dot(Lb_ref[rows, :], y_ref[...],
                    preferred_element_type=jnp.float32)
        out_ref[...] = a + acc_ref[rows, :].astype(jnp.float32)


def kernel(x, L_tilde, weight, bias):
    xb = x.astype(jnp.bfloat16)
    bias2d = bias.reshape(1, _F)

    out = pl.pallas_call(
        _cheb_kernel,
        grid=(_NBLK + _NOUT,),
        in_specs=[
            pl.BlockSpec(
                (_BM, _N),
                lambda m: (jnp.minimum(m, _NBLK - 1), 0)),
            pl.BlockSpec((_N, _F), lambda m: (0, 0)),
            pl.BlockSpec((3, _F, _F), lambda m: (0, 0, 0)),
            pl.BlockSpec((1, _F), lambda m: (0, 0)),
        ],
        out_specs=pl.BlockSpec(
            (_BO, _F),
            lambda m: (jnp.maximum(m - _NBLK, 0), 0)),
        out_shape=jax.ShapeDtypeStruct((_N, _F), jnp.float32),
        scratch_shapes=[
            pltpu.VMEM((_N, _N), jnp.bfloat16),
            pltpu.VMEM((_N, _F), jnp.bfloat16),
            pltpu.VMEM((_N, _F), jnp.bfloat16),
        ],
        compiler_params=pltpu.CompilerParams(
            dimension_semantics=("arbitrary",),
        ),
    )(L_tilde, xb, weight, bias2d)
    return out
